# baseline (speedup 1.0000x reference)
import functools
import math

import jax
import jax.numpy as jnp
from jax.experimental import pallas as pl
from jax.experimental.pallas import tpu as pltpu

H = 16
_PREC = jax.lax.Precision.DEFAULT
_NEG = -1e30


def _dot_t(a, b, prec=_PREC):
    return jax.lax.dot_general(a, b, (((1,), (1,)), ((), ())), precision=prec,
                               preferred_element_type=jnp.float32)


def _layernorm(x, w, b, eps=1e-5):
    mu = jnp.mean(x, axis=-1, keepdims=True)
    var = jnp.mean((x - mu) ** 2, axis=-1, keepdims=True)
    return (x - mu) / jnp.sqrt(var + eps) * w + b


def _attn_kernel(x_ref, wq_ref, wk_ref, wv_ref, wo_ref, gq_ref, bq_ref,
                 gk_ref, bk_ref, out_ref, f_ref, k_scr, v_scr, carry_ref,
                 *, bq, n, dh, scale, bw):
    qi = pl.program_id(0)

    @pl.when(qi == 0)
    def _():
        carry_ref[...] = jnp.zeros_like(carry_ref)

    x = x_ref[...]
    qblk = _layernorm(_dot_t(x, wq_ref[...]), gq_ref[...],
                      bq_ref[...]).astype(jnp.bfloat16)
    k_scr[pl.ds(qi * bq, bq), :] = _layernorm(
        _dot_t(x, wk_ref[...]), gk_ref[...], bk_ref[...]).astype(jnp.bfloat16)
    v_scr[pl.ds(qi * bq, bq), :] = _dot_t(x, wv_ref[...])

    ii = jax.lax.broadcasted_iota(jnp.int32, (bq, bq), 0)
    jj = jax.lax.broadcasted_iota(jnp.int32, (bq, bq), 1)
    ltri = (jj < ii).astype(jnp.float32)

    def _block(w):
        row = qi * bq + jax.lax.broadcasted_iota(jnp.int32, (bq, w), 0)
        col = jax.lax.broadcasted_iota(jnp.int32, (bq, w), 1)

        l0 = _dot_t(qblk[:, 0:dh], k_scr[0:w, 0:dh]) * scale
        s = jnp.where((col >= 1) & (col < row), jnp.maximum(l0, 0.0), 0.0)

        f_intra = jax.lax.dot_general(ltri, s, (((1,), (0,)), ((), ())),
                                      precision=_PREC)
        f = carry_ref[0:1, 0:w] + f_intra
        f_ref[:, 0:w] = f
        if w < n:
            f_ref[:, w:n] = jnp.zeros((bq, n - w), jnp.float32)
        carry_ref[0:1, 0:w] = carry_ref[0:1, 0:w] + jnp.sum(s, axis=0,
                                                            keepdims=True)

        base = jnp.where(col <= row, -f, _NEG)
        for h in range(H):
            if h == 0:
                lh = l0
            else:
                lh = _dot_t(qblk[:, h * dh:(h + 1) * dh],
                            k_scr[0:w, h * dh:(h + 1) * dh]) * scale
            p = jnp.exp(lh + base)
            denom = jnp.sum(p, axis=1, keepdims=True)
            oh = jax.lax.dot_general(p, v_scr[0:w, h * dh:(h + 1) * dh],
                                     (((1,), (0,)), ((), ())),
                                     precision=_PREC,
                                     preferred_element_type=jnp.float32)
            out_ref[:, h * dh:(h + 1) * dh] = oh / denom
        out_ref[...] = _dot_t(out_ref[...], wo_ref[...])

    nb = n // bw
    qpb = bw // bq
    for t in range(nb):
        w = bw * (t + 1)

        @pl.when((qi >= t * qpb) & (qi < (t + 1) * qpb))
        def _(w=w):
            _block(w)


def kernel(X, W_q, W_k, W_v, W_o, norm_q_w, norm_q_b, norm_k_w, norm_k_b,
           start_pos):
    b, n, d = X.shape
    dh = d // H
    scale = 1.0 / math.sqrt(dh)
    x2 = X.reshape(n, d)
    gq = norm_q_w.reshape(1, d)
    bq_ = norm_q_b.reshape(1, d)
    gk = norm_k_w.reshape(1, d)
    bk_ = norm_k_b.reshape(1, d)

    bqs = min(256, n)
    body = functools.partial(_attn_kernel, bq=bqs, n=n, dh=dh, scale=scale,
                             bw=max(bqs, n // 8))
    full = pl.BlockSpec((d, d), lambda i: (0, 0))
    vec = pl.BlockSpec((1, d), lambda i: (0, 0))
    out, f_mask = pl.pallas_call(
        body,
        grid=(n // bqs,),
        in_specs=[pl.BlockSpec((bqs, d), lambda i: (i, 0)),
                  full, full, full, full, vec, vec, vec, vec],
        out_specs=[pl.BlockSpec((bqs, d), lambda i: (i, 0)),
                   pl.BlockSpec((bqs, n), lambda i: (i, 0))],
        out_shape=[jax.ShapeDtypeStruct((n, d), jnp.float32),
                   jax.ShapeDtypeStruct((n, n), jnp.float32)],
        scratch_shapes=[pltpu.VMEM((n, d), jnp.bfloat16),
                        pltpu.VMEM((n, d), jnp.float32),
                        pltpu.VMEM((8, n), jnp.float32)],
        compiler_params=pltpu.CompilerParams(
            dimension_semantics=("arbitrary",)),
    )(x2, W_q, W_k, W_v, W_o, gq, bq_, gk, bk_)

    return (out.reshape(b, n, d), f_mask.reshape(b, n, n))


# speedup vs baseline: 4.2981x; 4.2981x over previous
import functools
import math

import jax
import jax.numpy as jnp
from jax.experimental import pallas as pl
from jax.experimental.pallas import tpu as pltpu

H = 16
_PREC = jax.lax.Precision.DEFAULT
_NEG = -1e30


def _dot_t(a, b, prec=_PREC):
    return jax.lax.dot_general(a, b, (((1,), (1,)), ((), ())), precision=prec,
                               preferred_element_type=jnp.float32)


def _layernorm(x, w, b, eps=1e-5):
    mu = jnp.mean(x, axis=-1, keepdims=True)
    var = jnp.mean((x - mu) ** 2, axis=-1, keepdims=True)
    return (x - mu) / jnp.sqrt(var + eps) * w + b


def _attn_kernel(x_ref, wq_ref, wk_ref, wv_ref, wo_ref, gq_ref, bq_ref,
                 gk_ref, bk_ref, out_ref, f_ref, k_scr, v_scr, carry_ref,
                 *, bq, n, dh, scale, bw):
    qi = pl.program_id(0)

    @pl.when(qi == 0)
    def _():
        carry_ref[...] = jnp.zeros_like(carry_ref)

    x = x_ref[...]
    qblk = _layernorm(_dot_t(x, wq_ref[...]), gq_ref[...],
                      bq_ref[...]).astype(jnp.bfloat16)
    k_scr[pl.ds(qi * bq, bq), :] = _layernorm(
        _dot_t(x, wk_ref[...]), gk_ref[...], bk_ref[...]).astype(jnp.bfloat16)
    v_scr[pl.ds(qi * bq, bq), :] = _dot_t(x, wv_ref[...])

    ii = jax.lax.broadcasted_iota(jnp.int32, (bq, bq), 0)
    jj = jax.lax.broadcasted_iota(jnp.int32, (bq, bq), 1)
    ltri = (jj < ii).astype(jnp.float32)

    def _block(w):
        row = qi * bq + jax.lax.broadcasted_iota(jnp.int32, (bq, w), 0)
        col = jax.lax.broadcasted_iota(jnp.int32, (bq, w), 1)

        l0 = _dot_t(qblk[:, 0:dh], k_scr[0:w, 0:dh]) * scale
        s = jnp.where((col >= 1) & (col < row), jnp.maximum(l0, 0.0), 0.0)

        f_intra = jax.lax.dot_general(ltri, s, (((1,), (0,)), ((), ())),
                                      precision=_PREC)
        f = carry_ref[0:1, 0:w] + f_intra
        f_ref[:, 0:w] = f
        if w < n:
            f_ref[:, w:n] = jnp.zeros((bq, n - w), jnp.float32)
        carry_ref[0:1, 0:w] = carry_ref[0:1, 0:w] + jnp.sum(s, axis=0,
                                                            keepdims=True)

        base = jnp.where(col <= row, -f, _NEG)
        for h in range(H):
            if h == 0:
                lh = l0
            else:
                lh = _dot_t(qblk[:, h * dh:(h + 1) * dh],
                            k_scr[0:w, h * dh:(h + 1) * dh]) * scale
            p = jnp.exp(lh + base)
            denom = jnp.sum(p, axis=1, keepdims=True)
            oh = jax.lax.dot_general(p, v_scr[0:w, h * dh:(h + 1) * dh],
                                     (((1,), (0,)), ((), ())),
                                     precision=_PREC,
                                     preferred_element_type=jnp.float32)
            out_ref[:, h * dh:(h + 1) * dh] = oh / denom
        out_ref[...] = _dot_t(out_ref[...], wo_ref[...])

    nb = n // bw
    qpb = bw // bq
    for t in range(nb):
        w = bw * (t + 1)

        @pl.when((qi >= t * qpb) & (qi < (t + 1) * qpb))
        def _(w=w):
            _block(w)


def kernel(X, W_q, W_k, W_v, W_o, norm_q_w, norm_q_b, norm_k_w, norm_k_b,
           start_pos):
    b, n, d = X.shape
    dh = d // H
    scale = 1.0 / math.sqrt(dh)
    x2 = X.reshape(n, d)
    gq = norm_q_w.reshape(1, d)
    bq_ = norm_q_b.reshape(1, d)
    gk = norm_k_w.reshape(1, d)
    bk_ = norm_k_b.reshape(1, d)

    bqs = min(256, n)
    body = functools.partial(_attn_kernel, bq=bqs, n=n, dh=dh, scale=scale,
                             bw=max(bqs, n // 4))
    full = pl.BlockSpec((d, d), lambda i: (0, 0))
    vec = pl.BlockSpec((1, d), lambda i: (0, 0))
    out, f_mask = pl.pallas_call(
        body,
        grid=(n // bqs,),
        in_specs=[pl.BlockSpec((bqs, d), lambda i: (i, 0)),
                  full, full, full, full, vec, vec, vec, vec],
        out_specs=[pl.BlockSpec((bqs, d), lambda i: (i, 0)),
                   pl.BlockSpec((bqs, n), lambda i: (i, 0))],
        out_shape=[jax.ShapeDtypeStruct((n, d), jnp.float32),
                   jax.ShapeDtypeStruct((n, n), jnp.float32)],
        scratch_shapes=[pltpu.VMEM((n, d), jnp.bfloat16),
                        pltpu.VMEM((n, d), jnp.float32),
                        pltpu.VMEM((8, n), jnp.float32)],
        compiler_params=pltpu.CompilerParams(
            dimension_semantics=("arbitrary",)),
    )(x2, W_q, W_k, W_v, W_o, gq, bq_, gk, bk_)

    return (out.reshape(b, n, d), f_mask.reshape(b, n, n))
